# Initial kernel scaffold; baseline (speedup 1.0000x reference)
import jax
import jax.numpy as jnp
from jax.experimental import pallas as pl


def kernel(x, edge_index, W1, b1, W2, b2, W3, b3):
    raise NotImplementedError("write your pallas kernel here")


# speedup vs baseline: 4.8100x; 4.8100x over previous
import functools

import jax
import jax.numpy as jnp
from jax import lax
from jax.experimental import pallas as pl
from jax.experimental.pallas import tpu as pltpu
from jax.experimental.pallas import tpu_sc as plsc

_N = 10000
_E = 320000
_NPAD = 10240
_D_IN = 128
_D_H = 128
_D3 = 128
_NC = 2
_NS = 16
_NW = _NC * _NS
_EPW = _E // _NW
_CH = 80
_NIT = _EPW // _CH
_RPT = _NPAD // _NS
_BLK = 1280


def _sc_mesh():
    return plsc.VectorSubcoreMesh(core_axis_name="c", subcore_axis_name="s")


@functools.partial(
    pl.kernel,
    out_type=jax.ShapeDtypeStruct((_NC, 2, _NPAD), jnp.float32),
    mesh=_sc_mesh(),
    scratch_types=[
        pltpu.VMEM((_CH,), jnp.int32),
        pltpu.VMEM((_CH,), jnp.float32),
        pltpu.VMEM((_RPT,), jnp.float32),
        pltpu.VMEM_SHARED((_NPAD,), jnp.float32),
        pltpu.VMEM_SHARED((_NPAD,), jnp.float32),
    ],
)
def _degree_hist(src_hbm, dst_hbm, out_hbm, idx_v, ones_v, zbuf, acc_s, acc_d):
    c = lax.axis_index("c")
    s = lax.axis_index("s")
    wid = c * _NS + s

    def _fill_z(i, _):
        zbuf[pl.ds(i * 16, 16)] = jnp.zeros((16,), jnp.float32)
        return 0

    lax.fori_loop(0, _RPT // 16, _fill_z, 0)

    def _fill_o(i, _):
        ones_v[pl.ds(i * 16, 16)] = jnp.ones((16,), jnp.float32)
        return 0

    lax.fori_loop(0, _CH // 16, _fill_o, 0)

    row0 = s * _RPT
    pltpu.sync_copy(zbuf, acc_s.at[pl.ds(row0, _RPT)])
    pltpu.sync_copy(zbuf, acc_d.at[pl.ds(row0, _RPT)])
    plsc.subcore_barrier()

    def _body(j, _):
        base = wid * _EPW + j * _CH
        pltpu.sync_copy(src_hbm.at[pl.ds(base, _CH)], idx_v)
        pltpu.sync_copy(ones_v, acc_s.at[idx_v], add=True)
        pltpu.sync_copy(dst_hbm.at[pl.ds(base, _CH)], idx_v)
        pltpu.sync_copy(ones_v, acc_d.at[idx_v], add=True)
        return 0

    lax.fori_loop(0, _NIT, _body, 0)

    plsc.subcore_barrier()
    pltpu.sync_copy(acc_s.at[pl.ds(row0, _RPT)], out_hbm.at[c, 0, pl.ds(row0, _RPT)])
    pltpu.sync_copy(acc_d.at[pl.ds(row0, _RPT)], out_hbm.at[c, 1, pl.ds(row0, _RPT)])


def _make_edge_pass(D):
    @functools.partial(
        pl.kernel,
        out_type=jax.ShapeDtypeStruct((_NC, _NPAD, D), jnp.float32),
        mesh=_sc_mesh(),
        scratch_types=[
            pltpu.VMEM((_CH,), jnp.int32),
            pltpu.VMEM((_CH,), jnp.int32),
            pltpu.VMEM((_CH, D), jnp.float32),
            pltpu.VMEM_SHARED((_NPAD, D), jnp.float32),
            pltpu.SemaphoreType.DMA,
        ],
    )
    def _edge_pass(t_hbm, src_hbm, dst_hbm, out_hbm, src_v, dst_v, rows_v, acc, sem):
        c = lax.axis_index("c")
        s = lax.axis_index("s")
        wid = c * _NS + s

        def _zrow(r, _):
            for k in range(D // 16):
                rows_v[r, pl.ds(k * 16, 16)] = jnp.zeros((16,), jnp.float32)
            return 0

        lax.fori_loop(0, _CH, _zrow, 0)
        row0 = s * _RPT
        for k in range(_RPT // _CH):
            pltpu.sync_copy(rows_v, acc.at[pl.ds(row0 + k * _CH, _CH)])
        plsc.subcore_barrier()

        def _body(j, _):
            base = wid * _EPW + j * _CH
            pltpu.sync_copy(src_hbm.at[pl.ds(base, _CH)], src_v)
            pltpu.sync_copy(dst_hbm.at[pl.ds(base, _CH)], dst_v)
            pltpu.async_copy(t_hbm.at[src_v], rows_v, sem).wait()
            pltpu.sync_copy(rows_v, acc.at[dst_v], add=True)
            return 0

        lax.fori_loop(0, _NIT, _body, 0)

        plsc.subcore_barrier()
        for k in range(_RPT // _CH):
            r = row0 + k * _CH
            pltpu.sync_copy(acc.at[pl.ds(r, _CH)], out_hbm.at[c, pl.ds(r, _CH)])

    return _edge_pass


_edge_pass_h = _make_edge_pass(_D_H)


def _stage_a_body(hist_ref, x_ref, w1_ref, ns_ref, nd_ref, t1_ref):
    h = hist_ref[...]
    deg_out = h[0, 0] + h[1, 0]
    deg_in = h[0, 1] + h[1, 1]
    ns = lax.rsqrt(jnp.maximum(deg_out, 1.0))
    nd = lax.rsqrt(jnp.maximum(deg_in, 1.0))
    ns_ref[...] = ns
    nd_ref[...] = nd
    t1_ref[...] = jnp.dot(x_ref[...] * ns, w1_ref[...],
                          preferred_element_type=jnp.float32)


def _stage_mid_body(p_ref, nd_ref, b_ref, hprev_ref, ns_ref, w_ref,
                    hnew_ref, tnext_ref):
    p = p_ref[...]
    agg = (p[0] + p[1]) * nd_ref[...] + b_ref[...]
    hnew = jnp.maximum(agg, 0.0) + hprev_ref[...]
    hnew_ref[...] = hnew
    tnext_ref[...] = jnp.dot(hnew * ns_ref[...], w_ref[...],
                             preferred_element_type=jnp.float32)


def _stage_d_body(p_ref, nd_ref, b_ref, out_ref):
    p = p_ref[...]
    out_ref[...] = (p[0, :, :40] + p[1, :, :40]) * nd_ref[...] + b_ref[...]


def _row_spec(d):
    return pl.BlockSpec((_BLK, d), lambda i: (i, 0))


def _stage_a(hist4, x_pad, W1):
    return pl.pallas_call(
        _stage_a_body,
        grid=(_NPAD // _BLK,),
        in_specs=[
            pl.BlockSpec((_NC, 2, _BLK, 1), lambda i: (0, 0, i, 0)),
            _row_spec(_D_IN),
            pl.BlockSpec((_D_IN, _D_H), lambda i: (0, 0)),
        ],
        out_specs=[_row_spec(1), _row_spec(1), _row_spec(_D_H)],
        out_shape=[
            jax.ShapeDtypeStruct((_NPAD, 1), jnp.float32),
            jax.ShapeDtypeStruct((_NPAD, 1), jnp.float32),
            jax.ShapeDtypeStruct((_NPAD, _D_H), jnp.float32),
        ],
    )(hist4, x_pad, W1)


def _stage_mid(P, nd, b, h_prev, ns, W_next, d_next):
    return pl.pallas_call(
        _stage_mid_body,
        grid=(_NPAD // _BLK,),
        in_specs=[
            pl.BlockSpec((_NC, _BLK, _D_H), lambda i: (0, i, 0)),
            _row_spec(1),
            pl.BlockSpec((1, _D_H), lambda i: (0, 0)),
            _row_spec(_D_H),
            _row_spec(1),
            pl.BlockSpec((_D_H, d_next), lambda i: (0, 0)),
        ],
        out_specs=[_row_spec(_D_H), _row_spec(d_next)],
        out_shape=[
            jax.ShapeDtypeStruct((_NPAD, _D_H), jnp.float32),
            jax.ShapeDtypeStruct((_NPAD, d_next), jnp.float32),
        ],
    )(P, nd, b, h_prev, ns, W_next)


def _stage_d(P, nd, b3):
    return pl.pallas_call(
        _stage_d_body,
        grid=(_NPAD // _BLK,),
        in_specs=[
            pl.BlockSpec((_NC, _BLK, _D3), lambda i: (0, i, 0)),
            _row_spec(1),
            pl.BlockSpec((1, 40), lambda i: (0, 0)),
        ],
        out_specs=_row_spec(40),
        out_shape=jax.ShapeDtypeStruct((_NPAD, 40), jnp.float32),
    )(P, nd, b3)


def kernel(x, edge_index, W1, b1, W2, b2, W3, b3):
    src = edge_index[0]
    dst = edge_index[1]
    x_pad = jnp.pad(x, ((0, _NPAD - _N), (0, 0)))
    W3p = jnp.pad(W3, ((0, 0), (0, _D3 - 40)))
    b1r = b1.reshape(1, _D_H)
    b2r = b2.reshape(1, _D_H)
    b3r = b3.reshape(1, 40)

    hist = _degree_hist(src, dst)
    hist4 = hist.reshape(_NC, 2, _NPAD, 1)
    ns, nd, t1 = _stage_a(hist4, x_pad, W1)

    P1 = _edge_pass_h(t1, src, dst)
    h1, t2 = _stage_mid(P1, nd, b1r, x_pad, ns, W2, _D_H)

    P2 = _edge_pass_h(t2, src, dst)
    _, t3 = _stage_mid(P2, nd, b2r, h1, ns, W3p, _D3)

    P3 = _edge_pass_h(t3, src, dst)
    out = _stage_d(P3, nd, b3r)
    return out[:_N]


# speedup vs baseline: 7.1722x; 1.4911x over previous
import functools

import jax
import jax.numpy as jnp
from jax import lax
from jax.experimental import pallas as pl
from jax.experimental.pallas import tpu as pltpu
from jax.experimental.pallas import tpu_sc as plsc

_N = 10000
_E = 320000
_NPAD = 10240
_D_IN = 128
_D_H = 128
_D3 = 128
_NC = 2
_NS = 16
_NW = _NC * _NS
_EPW = _E // _NW
_CH = 80
_NIT = _EPW // _CH
_RPT = _NPAD // _NS
_BLK = 1280


def _sc_mesh():
    return plsc.VectorSubcoreMesh(core_axis_name="c", subcore_axis_name="s")


@functools.partial(
    pl.kernel,
    out_type=jax.ShapeDtypeStruct((_NC, 2, _NPAD), jnp.float32),
    mesh=_sc_mesh(),
    scratch_types=[
        pltpu.VMEM((_CH,), jnp.int32),
        pltpu.VMEM((_CH,), jnp.float32),
        pltpu.VMEM((_RPT,), jnp.float32),
        pltpu.VMEM_SHARED((_NPAD,), jnp.float32),
        pltpu.VMEM_SHARED((_NPAD,), jnp.float32),
    ],
)
def _degree_hist(src_hbm, dst_hbm, out_hbm, idx_v, ones_v, zbuf, acc_s, acc_d):
    c = lax.axis_index("c")
    s = lax.axis_index("s")
    wid = c * _NS + s

    def _fill_z(i, _):
        zbuf[pl.ds(i * 16, 16)] = jnp.zeros((16,), jnp.float32)
        return 0

    lax.fori_loop(0, _RPT // 16, _fill_z, 0)

    def _fill_o(i, _):
        ones_v[pl.ds(i * 16, 16)] = jnp.ones((16,), jnp.float32)
        return 0

    lax.fori_loop(0, _CH // 16, _fill_o, 0)

    row0 = s * _RPT
    pltpu.sync_copy(zbuf, acc_s.at[pl.ds(row0, _RPT)])
    pltpu.sync_copy(zbuf, acc_d.at[pl.ds(row0, _RPT)])
    plsc.subcore_barrier()

    def _body(j, _):
        base = wid * _EPW + j * _CH
        pltpu.sync_copy(src_hbm.at[pl.ds(base, _CH)], idx_v)
        pltpu.sync_copy(ones_v, acc_s.at[idx_v], add=True)
        pltpu.sync_copy(dst_hbm.at[pl.ds(base, _CH)], idx_v)
        pltpu.sync_copy(ones_v, acc_d.at[idx_v], add=True)
        return 0

    lax.fori_loop(0, _NIT, _body, 0)

    plsc.subcore_barrier()
    pltpu.sync_copy(acc_s.at[pl.ds(row0, _RPT)], out_hbm.at[c, 0, pl.ds(row0, _RPT)])
    pltpu.sync_copy(acc_d.at[pl.ds(row0, _RPT)], out_hbm.at[c, 1, pl.ds(row0, _RPT)])


def _make_edge_pass(D):
    @functools.partial(
        pl.kernel,
        out_type=jax.ShapeDtypeStruct((_NC, _NPAD, D), jnp.float32),
        mesh=_sc_mesh(),
        scratch_types=[
            pltpu.VMEM((_CH,), jnp.int32),
            pltpu.VMEM((_CH,), jnp.int32),
            pltpu.VMEM((_CH,), jnp.int32),
            pltpu.VMEM((_CH,), jnp.int32),
            pltpu.VMEM((_CH, D), jnp.float32),
            pltpu.VMEM((_CH, D), jnp.float32),
            pltpu.VMEM_SHARED((_NPAD, D), jnp.float32),
            pltpu.SemaphoreType.DMA,
            pltpu.SemaphoreType.DMA,
        ],
    )
    def _edge_pass(t_hbm, src_hbm, dst_hbm, out_hbm, s0, d0, s1, d1, r0, r1,
                   acc, gsem, ssem):
        c = lax.axis_index("c")
        s = lax.axis_index("s")
        wid = c * _NS + s

        def _zrow(r, _):
            for k in range(D // 16):
                r0[r, pl.ds(k * 16, 16)] = jnp.zeros((16,), jnp.float32)
            return 0

        lax.fori_loop(0, _CH, _zrow, 0)
        row0 = s * _RPT
        for k in range(_RPT // _CH):
            pltpu.sync_copy(r0, acc.at[pl.ds(row0 + k * _CH, _CH)])
        plsc.subcore_barrier()

        def _load_idx(j, sv, dv):
            base = wid * _EPW + j * _CH
            pltpu.sync_copy(src_hbm.at[pl.ds(base, _CH)], sv)
            pltpu.sync_copy(dst_hbm.at[pl.ds(base, _CH)], dv)

        _load_idx(0, s0, d0)
        pltpu.async_copy(t_hbm.at[s0], r0, gsem)
        _load_idx(1, s1, d1)
        pltpu.make_async_copy(t_hbm.at[s0], r0, gsem).wait()
        pltpu.async_copy(r0, acc.at[d0], ssem, add=True)
        pltpu.async_copy(t_hbm.at[s1], r1, gsem)

        def _body(j, sa, da, ra, sb, db, rb, more):
            pltpu.make_async_copy(rb, acc.at[db], ssem).wait()

            def _prefetch():
                _load_idx(j + 1, sb, db)

            if more is True:
                _prefetch()
            else:
                pl.when(more)(_prefetch)
            pltpu.make_async_copy(t_hbm.at[sa], ra, gsem).wait()
            pltpu.async_copy(ra, acc.at[da], ssem, add=True)

            def _next_gather():
                pltpu.async_copy(t_hbm.at[sb], rb, gsem)

            if more is True:
                _next_gather()
            else:
                pl.when(more)(_next_gather)

        def _pair(i, _):
            j0 = 2 * i + 1
            j1 = j0 + 1
            _body(j0, s1, d1, r1, s0, d0, r0, True)
            _body(j1, s0, d0, r0, s1, d1, r1, j1 + 1 < _NIT)
            return 0

        lax.fori_loop(0, (_NIT - 1) // 2, _pair, 0)

        pltpu.make_async_copy(r0, acc.at[d0], ssem).wait()
        plsc.subcore_barrier()

        for k in range(_RPT // _CH):
            r = row0 + k * _CH
            pltpu.sync_copy(acc.at[pl.ds(r, _CH)], out_hbm.at[c, pl.ds(r, _CH)])

    return _edge_pass


_edge_pass_h = _make_edge_pass(_D_H)


def _stage_a_body(hist_ref, x_ref, w1_ref, ns_ref, nd_ref, t1_ref):
    h = hist_ref[...]
    deg_out = h[0, 0] + h[1, 0]
    deg_in = h[0, 1] + h[1, 1]
    ns = lax.rsqrt(jnp.maximum(deg_out, 1.0))
    nd = lax.rsqrt(jnp.maximum(deg_in, 1.0))
    ns_ref[...] = ns
    nd_ref[...] = nd
    t1_ref[...] = jnp.dot(x_ref[...] * ns, w1_ref[...],
                          preferred_element_type=jnp.float32)


def _stage_mid_body(p_ref, nd_ref, b_ref, hprev_ref, ns_ref, w_ref,
                    hnew_ref, tnext_ref):
    p = p_ref[...]
    agg = (p[0] + p[1]) * nd_ref[...] + b_ref[...]
    hnew = jnp.maximum(agg, 0.0) + hprev_ref[...]
    hnew_ref[...] = hnew
    tnext_ref[...] = jnp.dot(hnew * ns_ref[...], w_ref[...],
                             preferred_element_type=jnp.float32)


def _stage_d_body(p_ref, nd_ref, b_ref, out_ref):
    p = p_ref[...]
    out_ref[...] = (p[0, :, :40] + p[1, :, :40]) * nd_ref[...] + b_ref[...]


def _row_spec(d):
    return pl.BlockSpec((_BLK, d), lambda i: (i, 0))


def _stage_a(hist4, x_pad, W1):
    return pl.pallas_call(
        _stage_a_body,
        grid=(_NPAD // _BLK,),
        in_specs=[
            pl.BlockSpec((_NC, 2, _BLK, 1), lambda i: (0, 0, i, 0)),
            _row_spec(_D_IN),
            pl.BlockSpec((_D_IN, _D_H), lambda i: (0, 0)),
        ],
        out_specs=[_row_spec(1), _row_spec(1), _row_spec(_D_H)],
        out_shape=[
            jax.ShapeDtypeStruct((_NPAD, 1), jnp.float32),
            jax.ShapeDtypeStruct((_NPAD, 1), jnp.float32),
            jax.ShapeDtypeStruct((_NPAD, _D_H), jnp.float32),
        ],
    )(hist4, x_pad, W1)


def _stage_mid(P, nd, b, h_prev, ns, W_next, d_next):
    return pl.pallas_call(
        _stage_mid_body,
        grid=(_NPAD // _BLK,),
        in_specs=[
            pl.BlockSpec((_NC, _BLK, _D_H), lambda i: (0, i, 0)),
            _row_spec(1),
            pl.BlockSpec((1, _D_H), lambda i: (0, 0)),
            _row_spec(_D_H),
            _row_spec(1),
            pl.BlockSpec((_D_H, d_next), lambda i: (0, 0)),
        ],
        out_specs=[_row_spec(_D_H), _row_spec(d_next)],
        out_shape=[
            jax.ShapeDtypeStruct((_NPAD, _D_H), jnp.float32),
            jax.ShapeDtypeStruct((_NPAD, d_next), jnp.float32),
        ],
    )(P, nd, b, h_prev, ns, W_next)


def _stage_d(P, nd, b3):
    return pl.pallas_call(
        _stage_d_body,
        grid=(_NPAD // _BLK,),
        in_specs=[
            pl.BlockSpec((_NC, _BLK, _D3), lambda i: (0, i, 0)),
            _row_spec(1),
            pl.BlockSpec((1, 40), lambda i: (0, 0)),
        ],
        out_specs=_row_spec(40),
        out_shape=jax.ShapeDtypeStruct((_NPAD, 40), jnp.float32),
    )(P, nd, b3)


def kernel(x, edge_index, W1, b1, W2, b2, W3, b3):
    src = edge_index[0]
    dst = edge_index[1]
    x_pad = jnp.pad(x, ((0, _NPAD - _N), (0, 0)))
    W3p = jnp.pad(W3, ((0, 0), (0, _D3 - 40)))
    b1r = b1.reshape(1, _D_H)
    b2r = b2.reshape(1, _D_H)
    b3r = b3.reshape(1, 40)

    hist = _degree_hist(src, dst)
    hist4 = hist.reshape(_NC, 2, _NPAD, 1)
    ns, nd, t1 = _stage_a(hist4, x_pad, W1)

    P1 = _edge_pass_h(t1, src, dst)
    h1, t2 = _stage_mid(P1, nd, b1r, x_pad, ns, W2, _D_H)

    P2 = _edge_pass_h(t2, src, dst)
    _, t3 = _stage_mid(P2, nd, b2r, h1, ns, W3p, _D3)

    P3 = _edge_pass_h(t3, src, dst)
    out = _stage_d(P3, nd, b3r)
    return out[:_N]
